# Initial kernel scaffold; baseline (speedup 1.0000x reference)
import jax
import jax.numpy as jnp
from jax.experimental import pallas as pl


def kernel(enc, x_mark_enc, dec, k, v, att_weight, att_bias):
    raise NotImplementedError("write your pallas kernel here")


# speedup vs baseline: 1.1610x; 1.1610x over previous
import functools

import jax
import jax.numpy as jnp
from jax import lax
from jax.experimental import pallas as pl
from jax.experimental.pallas import tpu as pltpu
from jax.experimental.pallas import tpu_sc as plsc

B = 64
N = 883
T = 12
TN = 16
Q_BIAS = 0.1

NC = 2
NS = 16
NW = NC * NS

LN = 16
NPAD = 912
NCHUNK = 56
JOBS = NCHUNK * B
JPW = JOBS // NW
WIN = 48


def _tree_sum(xs):
    xs = list(xs)
    while len(xs) > 1:
        xs = [a + b for a, b in zip(xs[0::2], xs[1::2])] + (
            [xs[-1]] if len(xs) % 2 else [])
    return xs[0]


def _tree_max(xs):
    xs = list(xs)
    while len(xs) > 1:
        xs = [jnp.maximum(a, b) for a, b in zip(xs[0::2], xs[1::2])] + (
            [xs[-1]] if len(xs) % 2 else [])
    return xs[0]


def _rsqrt(x):
    xh = x * jnp.float32(0.5)
    i = lax.bitcast_convert_type(x, jnp.int32)
    i = jnp.int32(0x5F3759DF) - lax.shift_right_arithmetic(i, 1)
    y = lax.bitcast_convert_type(i, jnp.float32)
    for _ in range(3):
        y = y * (jnp.float32(1.5) - xh * y * y)
    return y


def _sqrt(x):
    return x * _rsqrt(jnp.maximum(x, jnp.float32(1e-30)))


def _sc_body(enc_hbm, dec_hbm, k_hbm, v_hbm, aw_hbm, ab_hbm, out_hbm,
             enc_v, dec_v, k_v, v_v, aw_v, ab_v):
    wid = lax.axis_index("s") * NC + lax.axis_index("c")
    job0 = wid * JPW
    c0 = lax.shift_right_logical(job0, 6)
    n_lo = c0 * LN

    pltpu.sync_copy(enc_hbm.at[:, :, pl.ds(n_lo, WIN)], enc_v)
    pltpu.sync_copy(dec_hbm.at[:, :, pl.ds(n_lo, WIN)], dec_v)
    pltpu.sync_copy(k_hbm.at[:, :, pl.ds(n_lo, WIN)], k_v)
    pltpu.sync_copy(v_hbm.at[:, :, pl.ds(n_lo, WIN)], v_v)
    pltpu.sync_copy(aw_hbm.at[:, pl.ds(n_lo, WIN)], aw_v)
    pltpu.sync_copy(ab_hbm.at[pl.ds(n_lo, WIN)], ab_v)

    cself = jnp.float32(0.12 ** 0.5)

    def body(i, carry):
        job = job0 + i
        chunk = lax.shift_right_logical(job, 6)
        b = job - chunk * B
        nloc = chunk * LN - n_lo
        ns = pl.ds(nloc, LN)

        q = [enc_v[b, j, ns] for j in range(T)]

        d = []
        for t in range(TN):
            df0 = q[0] - k_v[t, 0, ns]
            acc = df0 * df0
            for j in range(1, T):
                df = q[j] - k_v[t, j, ns]
                acc = acc + df * df
            d.append(_sqrt(acc))

        sum_d = _tree_sum(d)
        m1 = sum_d * jnp.float32(1.0 / TN)
        dev = [m1 - dt for dt in d]
        var1 = _tree_sum([x * x for x in dev]) * jnp.float32(1.0 / (TN - 1))
        std1 = _sqrt(var1) + jnp.float32(1e-6)
        coef1 = jnp.float32(10.0) / std1
        s1 = [x * coef1 for x in dev]
        mx1 = _tree_max(s1)
        e1 = [jnp.exp(x - mx1) for x in s1]
        inv_z1 = jnp.float32(1.0) / _tree_sum(e1)

        att = []
        for j in range(T):
            a = _tree_sum([e1[t] * v_v[t, j, ns] for t in range(TN)])
            att.append(a * inv_z1)

        m2 = (sum_d + cself) * jnp.float32(1.0 / (TN + 1))
        dev2 = [m2 - dt for dt in d]
        dev2c = m2 - cself
        var2 = (_tree_sum([x * x for x in dev2]) + dev2c * dev2c) * (
            jnp.float32(1.0 / TN))
        std2 = _sqrt(var2) + jnp.float32(1e-6)
        coef2 = jnp.float32(10.0) / std2
        s2 = [x * coef2 for x in dev2]
        s2c = dev2c * coef2
        mx2 = jnp.maximum(_tree_max(s2), s2c)
        e2 = [jnp.exp(x - mx2) for x in s2]
        e2c = jnp.exp(s2c - mx2)
        z2 = _tree_sum(e2) + e2c
        num = _tree_sum([e2[t] * aw_v[t, ns] for t in range(TN)])
        num = num + e2c * aw_v[TN, ns]
        w = num / z2 + ab_v[ns]

        for j in range(T):
            dj = dec_v[b, j, ns]
            dec_v[b, j, ns] = att[j] + w * (dj - att[j])

        pltpu.sync_copy(dec_v.at[b, :, pl.ds(nloc, LN)],
                        out_hbm.at[b, :, pl.ds(chunk * LN, LN)])
        return carry

    lax.fori_loop(0, JPW, body, 0)


@functools.partial(jax.jit, static_argnums=())
def _run_sc(enc_t, dec_t, k_t, v_t, aw_t, ab_t):
    mesh = plsc.VectorSubcoreMesh(
        core_axis_name="c", subcore_axis_name="s",
        num_cores=NC, num_subcores=NS)
    f = pl.kernel(
        _sc_body,
        out_type=jax.ShapeDtypeStruct((B, T, NPAD), jnp.float32),
        mesh=mesh,
        compiler_params=pltpu.CompilerParams(use_tc_tiling_on_sc=False),
        scratch_types=[
            pltpu.VMEM((B, T, WIN), jnp.float32),
            pltpu.VMEM((B, T, WIN), jnp.float32),
            pltpu.VMEM((TN, T, WIN), jnp.float32),
            pltpu.VMEM((TN, T, WIN), jnp.float32),
            pltpu.VMEM((TN + 1, WIN), jnp.float32),
            pltpu.VMEM((WIN,), jnp.float32),
        ],
    )
    return f(enc_t, dec_t, k_t, v_t, aw_t, ab_t)


def kernel(enc, x_mark_enc, dec, k, v, att_weight, att_bias):
    del x_mark_enc
    pad = NPAD - N
    enc_t = jnp.pad(jnp.transpose(enc, (0, 2, 1)), ((0, 0), (0, 0), (0, pad)))
    dec_t = jnp.pad(jnp.transpose(dec, (0, 2, 1)), ((0, 0), (0, 0), (0, pad)))
    k_t = jnp.pad(jnp.transpose(jnp.squeeze(k, 1), (1, 2, 0)),
                  ((0, 0), (0, 0), (0, pad)))
    v_t = jnp.pad(jnp.transpose(jnp.squeeze(v, 1), (1, 2, 0)),
                  ((0, 0), (0, 0), (0, pad)))
    aw_t = jnp.pad(att_weight.T, ((0, 0), (0, pad)))
    ab_t = jnp.pad(att_bias, ((0, pad)))
    out_t = _run_sc(enc_t, dec_t, k_t, v_t, aw_t, ab_t)
    return jnp.transpose(out_t[:, :, :N], (0, 2, 1))
